# baseline (speedup 1.0000x reference)
import jax
import jax.numpy as jnp
import numpy as np
from jax.experimental import pallas as pl

_N_TILES = 8
_L = 8
_NSQ = _N_TILES * _N_TILES
_OUT_COLS = _L * _NSQ

_REPL = np.repeat(np.eye(_L, dtype=np.float32), _NSQ, axis=1)
_BINID = (np.arange(_OUT_COLS, dtype=np.int64) % _NSQ).astype(np.float32)[None, :]


def _tyler_body(x_ref, t_ref, o_ref):
    x = x_ref[:, 0:1]
    y = x_ref[:, 1:2]
    tx = t_ref[0:1, :]
    ty = t_ref[1:2, :]
    dxt = jnp.float32(1.2)
    n = jnp.float32(_N_TILES)
    ix = ((x - tx) / dxt * n).astype(jnp.int32)
    iy = ((y - ty) / dxt * n).astype(jnp.int32)
    idx = ix + iy * _N_TILES
    idxb = idx.astype(jnp.bfloat16)
    lane = jax.lax.broadcasted_iota(jnp.int32, (_L, _OUT_COLS), 1)
    row = jax.lax.broadcasted_iota(jnp.int32, (_L, _OUT_COLS), 0)
    rep = ((lane >> 6) == row).astype(jnp.bfloat16)
    bcast = jax.lax.dot_general(
        idxb, rep, (((1,), (0,)), ((), ())),
        preferred_element_type=jnp.float32,
    )
    binid = (
        jax.lax.broadcasted_iota(jnp.int32, (1, _OUT_COLS), 1) & (_NSQ - 1)
    ).astype(jnp.float32)
    o_ref[:, :] = (bcast == binid).astype(jnp.float32)


@jax.jit
def kernel(x, tile0):
    n_points = x.shape[0]
    block = 10000
    grid = (pl.cdiv(n_points, block),)
    t_t = tile0.T
    return pl.pallas_call(
        _tyler_body,
        grid=grid,
        in_specs=[
            pl.BlockSpec((block, 2), lambda i: (i, 0)),
            pl.BlockSpec((2, _L), lambda i: (0, 0)),
        ],
        out_specs=pl.BlockSpec((block, _OUT_COLS), lambda i: (i, 0)),
        out_shape=jax.ShapeDtypeStruct((n_points, _OUT_COLS), jnp.float32),
    )(x, t_t)


# speedup vs baseline: 1.0081x; 1.0081x over previous
import jax
import jax.numpy as jnp
import numpy as np
from jax.experimental import pallas as pl
from jax.experimental.pallas import tpu as pltpu

_N_TILES = 8
_L = 8
_NSQ = _N_TILES * _N_TILES
_OUT_COLS = _L * _NSQ

_REPL = np.repeat(np.eye(_L, dtype=np.float32), _NSQ, axis=1)
_BINID = (np.arange(_OUT_COLS, dtype=np.int64) % _NSQ).astype(np.float32)[None, :]


def _tyler_body(x_ref, t_ref, o_ref):
    x = x_ref[:, 0:1]
    y = x_ref[:, 1:2]
    tx = t_ref[0:1, :]
    ty = t_ref[1:2, :]
    dxt = jnp.float32(1.2)
    n = jnp.float32(_N_TILES)
    ix = ((x - tx) / dxt * n).astype(jnp.int32)
    iy = ((y - ty) / dxt * n).astype(jnp.int32)
    idx = ix + iy * _N_TILES
    idxb = idx.astype(jnp.bfloat16)
    lane = jax.lax.broadcasted_iota(jnp.int32, (_L, _OUT_COLS), 1)
    row = jax.lax.broadcasted_iota(jnp.int32, (_L, _OUT_COLS), 0)
    rep = ((lane >> 6) == row).astype(jnp.bfloat16)
    bcast = jax.lax.dot_general(
        idxb, rep, (((1,), (0,)), ((), ())),
        preferred_element_type=jnp.float32,
    )
    binid = (
        jax.lax.broadcasted_iota(jnp.int32, (1, _OUT_COLS), 1) & (_NSQ - 1)
    ).astype(jnp.float32)
    o_ref[:, :] = (bcast == binid).astype(jnp.float32)


@jax.jit
def kernel(x, tile0):
    n_points = x.shape[0]
    block = 5000
    grid = (pl.cdiv(n_points, block),)
    t_t = tile0.T
    return pl.pallas_call(
        _tyler_body,
        grid=grid,
        in_specs=[
            pl.BlockSpec((block, 2), lambda i: (i, 0)),
            pl.BlockSpec((2, _L), lambda i: (0, 0)),
        ],
        out_specs=pl.BlockSpec((block, _OUT_COLS), lambda i: (i, 0)),
        out_shape=jax.ShapeDtypeStruct((n_points, _OUT_COLS), jnp.float32),
        compiler_params=pltpu.CompilerParams(
            dimension_semantics=("parallel",),
        ),
    )(x, t_t)
